# baseline (speedup 1.0000x reference)
import jax
import jax.numpy as jnp
from jax import lax
from jax.experimental import pallas as pl
from jax.experimental.pallas import tpu as pltpu
from jax.experimental.pallas import tpu_sc as plsc

B, T, C, D = 4, 8192, 32, 496
NC, NS = 2, 16
NW = NC * NS
TOK = B * T
TPW = TOK // NW
CHUNK = 128
NCHUNK = TPW // CHUNK
NTG = CHUNK // 16
DSPLIT = 240


OSEG = ((0, 120), (120, 120), (240, 128), (368, 128))


def _body(x_hbm, w1_hbm, w2_hbm, out_hbm,
          xbuf0, xbuf1, xs, ob, w1b, w2b,
          insem0, insem1, osem0, osem1, osem2, osem3):
    wid = lax.axis_index("s") * NC + lax.axis_index("c")
    base = wid * TPW
    pltpu.sync_copy(w1_hbm, w1b)
    pltpu.sync_copy(w2_hbm, w2b)
    third = jnp.float32(1.0 / 3.0)
    i16 = lax.iota(jnp.int32, 16)
    xbufs = [xbuf0, xbuf1]
    insems = [insem0, insem1]

    def in_slices(k):
        t0 = base + k * CHUNK
        bb = lax.div(t0, T)
        tt = pl.multiple_of(lax.rem(t0, T), CHUNK)
        left = pl.multiple_of(jnp.where(tt == 0, T - CHUNK, tt - CHUNK),
                              CHUNK)
        right = pl.multiple_of(jnp.where(tt + CHUNK == T, 0, tt + CHUNK),
                               CHUNK)
        return bb, (left, tt, right)

    def fire_input(k, xb, sem):
        bb, cols = in_slices(k)
        for blk in range(3):
            pltpu.async_copy(x_hbm.at[bb, :, pl.ds(cols[blk], CHUNK)],
                             xb.at[blk], sem)

    def drain_input(k, xb, sem):
        bb, cols = in_slices(k)
        for blk in range(3):
            pltpu.make_async_copy(x_hbm.at[bb, :, pl.ds(cols[blk], CHUNK)],
                                  xb.at[blk], sem).wait()

    osems = [osem0, osem1, osem2, osem3]

    def wait_oseg(q):
        lo, n = OSEG[q]
        pltpu.make_async_copy(ob.at[pl.ds(lo, n), :],
                              out_hbm.at[0, pl.ds(lo, n), pl.ds(0, CHUNK)],
                              osems[q]).wait()

    def fire_oseg(q, bb, tt):
        lo, n = OSEG[q]
        pltpu.async_copy(ob.at[pl.ds(lo, n), :],
                         out_hbm.at[bb, pl.ds(lo, n), pl.ds(tt, CHUNK)],
                         osems[q])

    fire_input(0, xbuf0, insem0)

    def chunk2(k2, carry):
        for ph in range(2):
            k = k2 * 2 + ph
            t0 = base + k * CHUNK
            bb = lax.div(t0, T)
            tt = pl.multiple_of(lax.rem(t0, T), CHUNK)
            xb = xbufs[ph]
            drain_input(k, xb, insems[ph])
            fire_input(lax.rem(k + 1, NCHUNK), xbufs[1 - ph], insems[1 - ph])

            zero = lax.broadcast(0, (16,))
            one = lax.broadcast(1, (16,))
            two = lax.broadcast(2, (16,))

            @plsc.parallel_loop(0, C * NTG, unroll=2)
            def _smooth(j, xb=xb, zero=zero, one=one, two=two):
                c = lax.shift_right_logical(j, 3)
                tg = lax.bitwise_and(j, 7) * 16
                lanepos = lax.broadcast(tg, (16,)) + i16
                cv = lax.broadcast(c, (16,))
                pcol = lax.bitwise_and(lanepos - 1, CHUNK - 1)
                pblk = jnp.where(lanepos == 0, zero, one)
                ncol = lax.bitwise_and(lanepos + 1, CHUNK - 1)
                nblk = jnp.where(lanepos == CHUNK - 1, two, one)
                vp = plsc.load_gather(xb, [pblk, cv, pcol])
                vn = plsc.load_gather(xb, [nblk, cv, ncol])
                vc = xb[1, c, pl.ds(pl.multiple_of(tg, 16), 16)]
                xs[c, pl.ds(pl.multiple_of(tg, 16), 16)] = (
                    vp + vc + vn) * third

            def run_block(c1, dstart, lo, hi):
                bases = [xs[c1, pl.ds(tg * 16, 16)] for tg in range(NTG)]

                @plsc.parallel_loop(lo, hi, unroll=2)
                def _blk(dr, c1=c1, dstart=dstart, bases=bases):
                    dd = dstart + dr
                    woff = pl.multiple_of(dd * 16, 16)
                    w1v = w1b[pl.ds(woff, 16)]
                    w2v = w2b[pl.ds(woff, 16)]
                    c2r = c1 + 1 + dr
                    for tg in range(NTG):
                        a = xs[c2r, pl.ds(tg * 16, 16)]
                        ob[dd, pl.ds(tg * 16, 16)] = (bases[tg] * w1v
                                                      + a * w2v)

            segments = [
                (0, [(0, 0, 0, 31), (1, 31, 0, 30), (2, 61, 0, 29),
                     (3, 90, 0, 28), (4, 118, 0, 2)]),
                (1, [(4, 118, 2, 27), (5, 145, 0, 26), (6, 171, 0, 25),
                     (7, 196, 0, 24), (8, 220, 0, 20)]),
                (2, [(8, 220, 20, 23), (9, 243, 0, 22), (10, 265, 0, 21),
                     (11, 286, 0, 20), (12, 306, 0, 19), (13, 325, 0, 18),
                     (14, 343, 0, 17), (15, 360, 0, 8)]),
                (3, [(15, 360, 8, 16)]
                    + [(c1, s, 0, 31 - c1) for c1, s in
                       [(16, 376), (17, 391), (18, 405), (19, 418),
                        (20, 430), (21, 441), (22, 451), (23, 460),
                        (24, 468), (25, 475), (26, 481), (27, 486),
                        (28, 490), (29, 493), (30, 495)]]),
            ]
            for q, blocks in segments:
                @pl.when(k > 0)
                def _wait_q(q=q):
                    wait_oseg(q)
                for c1, dstart, lo, hi in blocks:
                    run_block(c1, dstart, lo, hi)
                fire_oseg(q, bb, tt)
        return carry

    lax.fori_loop(0, NCHUNK // 2, chunk2, 0)
    drain_input(0, xbuf0, insem0)
    for q in range(4):
        wait_oseg(q)


def kernel(x, weights, pairs_idx):
    del pairs_idx
    w1b = jnp.repeat(weights[:, 0], 16)
    w2b = jnp.repeat(weights[:, 1], 16)
    mesh = plsc.VectorSubcoreMesh(core_axis_name="c", subcore_axis_name="s")
    f = pl.kernel(
        _body,
        mesh=mesh,
        compiler_params=pltpu.CompilerParams(needs_layout_passes=False),
        out_type=jax.ShapeDtypeStruct((B, D, T), jnp.float32),
        scratch_types=[
            pltpu.VMEM((3, C, CHUNK), jnp.float32),
            pltpu.VMEM((3, C, CHUNK), jnp.float32),
            pltpu.VMEM((C, CHUNK), jnp.float32),
            pltpu.VMEM((D, CHUNK), jnp.float32),
            pltpu.VMEM((D * 16,), jnp.float32),
            pltpu.VMEM((D * 16,), jnp.float32),
            pltpu.SemaphoreType.DMA,
            pltpu.SemaphoreType.DMA,
            pltpu.SemaphoreType.DMA,
            pltpu.SemaphoreType.DMA,
            pltpu.SemaphoreType.DMA,
            pltpu.SemaphoreType.DMA,
        ],
    )
    xt = jnp.transpose(x, (0, 2, 1))
    out_bdt = f(xt, w1b, w2b)
    return jnp.transpose(out_bdt, (0, 2, 1))


# speedup vs baseline: 1.0293x; 1.0293x over previous
import jax
import jax.numpy as jnp
from jax import lax
from jax.experimental import pallas as pl
from jax.experimental.pallas import tpu as pltpu
from jax.experimental.pallas import tpu_sc as plsc

B, T, C, D = 4, 8192, 32, 496
NC, NS = 2, 16
NW = NC * NS
TOK = B * T
TPW = TOK // NW
CHUNK = 128
NCHUNK = TPW // CHUNK
NTG = CHUNK // 16
DSPLIT = 240


def _body(x_hbm, w1_hbm, w2_hbm, out_hbm,
          xbuf0, xbuf1, xs, ob, w1b, w2b,
          insem0, insem1, osema, osemb):
    wid = lax.axis_index("s") * NC + lax.axis_index("c")
    base = wid * TPW
    pltpu.sync_copy(w1_hbm, w1b)
    pltpu.sync_copy(w2_hbm, w2b)
    third = jnp.float32(1.0 / 3.0)
    i16 = lax.iota(jnp.int32, 16)
    xbufs = [xbuf0, xbuf1]
    insems = [insem0, insem1]

    def in_slices(k):
        t0 = base + k * CHUNK
        bb = lax.div(t0, T)
        tt = pl.multiple_of(lax.rem(t0, T), CHUNK)
        left = pl.multiple_of(jnp.where(tt == 0, T - CHUNK, tt - CHUNK),
                              CHUNK)
        right = pl.multiple_of(jnp.where(tt + CHUNK == T, 0, tt + CHUNK),
                               CHUNK)
        return bb, (left, tt, right)

    def fire_input(k, xb, sem):
        bb, cols = in_slices(k)
        for blk in range(3):
            pltpu.async_copy(x_hbm.at[bb, :, pl.ds(cols[blk], CHUNK)],
                             xb.at[blk], sem)

    def drain_input(k, xb, sem):
        bb, cols = in_slices(k)
        for blk in range(3):
            pltpu.make_async_copy(x_hbm.at[bb, :, pl.ds(cols[blk], CHUNK)],
                                  xb.at[blk], sem).wait()

    fire_input(0, xbuf0, insem0)

    def chunk2(k2, carry):
        for ph in range(2):
            k = k2 * 2 + ph
            t0 = base + k * CHUNK
            bb = lax.div(t0, T)
            tt = pl.multiple_of(lax.rem(t0, T), CHUNK)
            xb = xbufs[ph]
            drain_input(k, xb, insems[ph])
            fire_input(lax.rem(k + 1, NCHUNK), xbufs[1 - ph], insems[1 - ph])

            zero = lax.broadcast(0, (16,))
            one = lax.broadcast(1, (16,))
            two = lax.broadcast(2, (16,))

            @plsc.parallel_loop(0, C * NTG, unroll=2)
            def _smooth(j, xb=xb, zero=zero, one=one, two=two):
                c = lax.shift_right_logical(j, 3)
                tg = lax.bitwise_and(j, 7) * 16
                lanepos = lax.broadcast(tg, (16,)) + i16
                cv = lax.broadcast(c, (16,))
                pcol = lax.bitwise_and(lanepos - 1, CHUNK - 1)
                pblk = jnp.where(lanepos == 0, zero, one)
                ncol = lax.bitwise_and(lanepos + 1, CHUNK - 1)
                nblk = jnp.where(lanepos == CHUNK - 1, two, one)
                vp = plsc.load_gather(xb, [pblk, cv, pcol])
                vn = plsc.load_gather(xb, [nblk, cv, ncol])
                vc = xb[1, c, pl.ds(pl.multiple_of(tg, 16), 16)]
                xs[c, pl.ds(pl.multiple_of(tg, 16), 16)] = (
                    vp + vc + vn) * third

            @pl.when(k > 0)
            def _wait_a():
                pltpu.make_async_copy(
                    ob.at[pl.ds(0, DSPLIT), :],
                    out_hbm.at[0, pl.ds(0, DSPLIT), pl.ds(0, CHUNK)],
                    osema).wait()

            def run_block(c1, dstart, lo, hi):
                bases = [xs[c1, pl.ds(tg * 16, 16)] for tg in range(NTG)]

                @plsc.parallel_loop(lo, hi, unroll=2)
                def _blk(dr, c1=c1, dstart=dstart, bases=bases):
                    dd = dstart + dr
                    woff = pl.multiple_of(dd * 16, 16)
                    w1v = w1b[pl.ds(woff, 16)]
                    w2v = w2b[pl.ds(woff, 16)]
                    c2r = c1 + 1 + dr
                    for tg in range(NTG):
                        a = xs[c2r, pl.ds(tg * 16, 16)]
                        ob[dd, pl.ds(tg * 16, 16)] = (bases[tg] * w1v
                                                      + a * w2v)

            dstart = 0
            for c1 in range(9):
                blk_len = 31 - c1
                run_block(c1, dstart, 0, blk_len if c1 < 8 else 20)
                dstart += blk_len
            pltpu.async_copy(ob.at[pl.ds(0, DSPLIT), :],
                             out_hbm.at[bb, pl.ds(0, DSPLIT),
                                        pl.ds(tt, CHUNK)],
                             osema)

            @pl.when(k > 0)
            def _wait_b():
                pltpu.make_async_copy(
                    ob.at[pl.ds(DSPLIT, D - DSPLIT), :],
                    out_hbm.at[0, pl.ds(DSPLIT, D - DSPLIT), pl.ds(0, CHUNK)],
                    osemb).wait()

            run_block(8, 220, 20, 23)
            dstart = 243
            for c1 in range(9, 31):
                blk_len = 31 - c1
                run_block(c1, dstart, 0, blk_len)
                dstart += blk_len
            pltpu.async_copy(ob.at[pl.ds(DSPLIT, D - DSPLIT), :],
                             out_hbm.at[bb, pl.ds(DSPLIT, D - DSPLIT),
                                        pl.ds(tt, CHUNK)],
                             osemb)
        return carry

    lax.fori_loop(0, NCHUNK // 2, chunk2, 0)
    drain_input(0, xbuf0, insem0)
    pltpu.make_async_copy(ob.at[pl.ds(0, DSPLIT), :],
                          out_hbm.at[0, pl.ds(0, DSPLIT), pl.ds(0, CHUNK)],
                          osema).wait()
    pltpu.make_async_copy(ob.at[pl.ds(DSPLIT, D - DSPLIT), :],
                          out_hbm.at[0, pl.ds(DSPLIT, D - DSPLIT),
                                     pl.ds(0, CHUNK)],
                          osemb).wait()


def kernel(x, weights, pairs_idx):
    del pairs_idx
    w1b = jnp.repeat(weights[:, 0], 16)
    w2b = jnp.repeat(weights[:, 1], 16)
    mesh = plsc.VectorSubcoreMesh(core_axis_name="c", subcore_axis_name="s")
    f = pl.kernel(
        _body,
        mesh=mesh,
        compiler_params=pltpu.CompilerParams(needs_layout_passes=False),
        out_type=jax.ShapeDtypeStruct((B, D, T), jnp.float32),
        scratch_types=[
            pltpu.VMEM((3, C, CHUNK), jnp.float32),
            pltpu.VMEM((3, C, CHUNK), jnp.float32),
            pltpu.VMEM((C, CHUNK), jnp.float32),
            pltpu.VMEM((D, CHUNK), jnp.float32),
            pltpu.VMEM((D * 16,), jnp.float32),
            pltpu.VMEM((D * 16,), jnp.float32),
            pltpu.SemaphoreType.DMA,
            pltpu.SemaphoreType.DMA,
            pltpu.SemaphoreType.DMA,
            pltpu.SemaphoreType.DMA,
        ],
    )
    xt = jnp.transpose(x, (0, 2, 1))
    out_bdt = f(xt, w1b, w2b)
    return jnp.transpose(out_bdt, (0, 2, 1))
